# baseline (speedup 1.0000x reference)
import functools

import jax
import jax.numpy as jnp
from jax import lax
from jax.experimental import pallas as pl
from jax.experimental.pallas import tpu as pltpu
from jax.experimental.pallas import tpu_sc as plsc

N_NODES = 10000
N_EDGES = 320000
D_FEAT = 128
N_CLASSES = 64

NW = 32
RNG = 320
NPAD = NW * RNG
DUMP = RNG
ACC_ROWS = RNG + 8
CW = 16
CHUNK = 1600
NCHUNK = N_EDGES // CHUNK
SCAN_IT = CHUNK // 16
CAP = 272
SUB = 32
NSUB = CAP // SUB + 1
CROWS = RNG // 16 + 4


def _sc_aggregate(x_pad, src, dst):
    mesh = plsc.VectorSubcoreMesh(core_axis_name="c", subcore_axis_name="s")

    @functools.partial(
        pl.kernel,
        out_type=[
            jax.ShapeDtypeStruct((NPAD, D_FEAT), jnp.float32),
            jax.ShapeDtypeStruct((NPAD, D_FEAT), jnp.float32),
            jax.ShapeDtypeStruct((NW * CROWS, 16), jnp.float32),
        ],
        mesh=mesh,
        compiler_params=pltpu.CompilerParams(needs_layout_passes=False),
        scratch_types=[
            pltpu.VMEM((CHUNK,), jnp.int32),
            pltpu.VMEM((CHUNK,), jnp.int32),
            pltpu.VMEM((CAP,), jnp.int32),
            pltpu.VMEM((CAP,), jnp.int32),
            pltpu.VMEM((SUB,), jnp.int32),
            pltpu.VMEM((SUB, D_FEAT), jnp.float32),
            pltpu.VMEM((ACC_ROWS, D_FEAT), jnp.float32),
            pltpu.VMEM((CROWS, CW), jnp.float32),
            pltpu.VMEM_SHARED((16 * ACC_ROWS, D_FEAT), jnp.float32),
        ],
    )
    def body(x_hbm, src_hbm, dst_hbm, amax_hbm, asum_hbm, cnt_hbm,
             dstbuf, srcbuf, dl_list, src_list, idxv, rowbuf,
             accmax, acccnt, accsum):
        s_idx = lax.axis_index("s")
        w = s_idx * 2 + lax.axis_index("c")
        lo = w * RNG
        sbase = s_idx * ACC_ROWS
        neg = jnp.full((16,), -jnp.inf, dtype=jnp.float32)
        zero = jnp.zeros((16,), dtype=jnp.float32)
        lane = lax.broadcasted_iota(jnp.int32, (16,), 0)

        def init_row(i, _):
            for b in range(D_FEAT // 16):
                accmax[i, pl.ds(b * 16, 16)] = neg
            return 0

        lax.fori_loop(0, ACC_ROWS, init_row, 0)

        def init_cnt(i, _):
            acccnt[i, pl.ds(0, 16)] = zero
            return 0

        lax.fori_loop(0, CROWS, init_cnt, 0)

        def zero_rowbuf(i, _):
            for b in range(D_FEAT // 16):
                rowbuf[i, pl.ds(b * 16, 16)] = zero
            return 0

        lax.fori_loop(0, SUB, zero_rowbuf, 0)
        for r in range(ACC_ROWS // SUB):
            pltpu.sync_copy(rowbuf, accsum.at[pl.ds(sbase + r * SUB, SUB)])
        _rem = ACC_ROWS % SUB
        if _rem:
            pltpu.sync_copy(rowbuf.at[pl.ds(0, _rem)],
                            accsum.at[pl.ds(sbase + ACC_ROWS - _rem, _rem)])

        def do_chunk(k, _):
            pltpu.sync_copy(dst_hbm.at[pl.ds(k * CHUNK, CHUNK)], dstbuf)
            pltpu.sync_copy(src_hbm.at[pl.ds(k * CHUNK, CHUNK)], srcbuf)

            dumpv = jnp.full((16,), DUMP, dtype=jnp.int32)
            zi = jnp.zeros((16,), dtype=jnp.int32)

            def reset(i, _):
                dl_list[pl.ds(i * 16, 16)] = dumpv
                src_list[pl.ds(i * 16, 16)] = zi
                return 0

            lax.fori_loop(0, CAP // 16, reset, 0)

            def scan(i, off):
                d = dstbuf[pl.ds(i * 16, 16)]
                s = srcbuf[pl.ds(i * 16, 16)]
                msk = (d >= lo) & (d < lo + RNG)
                cs = plsc.cumsum(jnp.where(msk, 1, 0).astype(jnp.int32))
                pos = off + cs - 1
                plsc.store_scatter(dl_list, [pos], d - lo, mask=msk)
                plsc.store_scatter(src_list, [pos], s, mask=msk)
                cnt = plsc.all_reduce_population_count(msk)
                return jnp.minimum(off + cnt[0], CAP - 16)

            m = lax.fori_loop(0, SCAN_IT, scan, jnp.int32(0))

            def do_sub(c, _):
                @pl.when(m > c * SUB)
                def _():
                    sl = pl.ds(c * SUB, SUB)
                    pltpu.sync_copy(x_hbm.at[src_list.at[sl]], rowbuf)

                    def mk_idx(j, _):
                        dl16 = dl_list[pl.ds(c * SUB + j * 16, 16)]
                        idxv[pl.ds(j * 16, 16)] = dl16 + sbase
                        return 0

                    lax.fori_loop(0, SUB // 16, mk_idx, 0)
                    pltpu.sync_copy(rowbuf, accsum.at[idxv], add=True)

                    rem = jnp.minimum(m - c * SUB, SUB)
                    ng = (rem + 15) // 16

                    def emax(g, _):
                        dlv = dl_list[pl.ds(c * SUB + g * 16, 16)]
                        for j in range(16):
                            dl = dlv[j]
                            e = g * 16 + j
                            for b in range(D_FEAT // 16):
                                v = rowbuf[e, pl.ds(b * 16, 16)]
                                a = accmax[dl, pl.ds(b * 16, 16)]
                                accmax[dl, pl.ds(b * 16, 16)] = jnp.maximum(a, v)
                            rq = lax.shift_right_logical(dl, 4)
                            onev = jnp.where(
                                lane == jnp.bitwise_and(dl, 15), 1.0, 0.0
                            ).astype(jnp.float32)
                            acccnt[rq, pl.ds(0, 16)] = (
                                acccnt[rq, pl.ds(0, 16)] + onev)
                        return 0

                    lax.fori_loop(0, ng, emax, 0)
                return 0

            lax.fori_loop(0, NSUB, do_sub, 0)
            return 0

        lax.fori_loop(0, NCHUNK, do_chunk, 0)

        pltpu.sync_copy(accmax.at[pl.ds(0, RNG)], amax_hbm.at[pl.ds(lo, RNG)])
        pltpu.sync_copy(accsum.at[pl.ds(sbase, RNG)], asum_hbm.at[pl.ds(lo, RNG)])
        pltpu.sync_copy(acccnt.at[pl.ds(0, CROWS)],
                        cnt_hbm.at[pl.ds(w * CROWS, CROWS)])

    return body(x_pad, src, dst)


def _tc_head(amax, asum, cnt, x_pad, WlmaxT, WlmeanT, WrT, bias):
    def body(am_ref, as_ref, c_ref, x_ref, wlx_ref, wlm_ref, wr_ref, b_ref,
             o_ref):
        am = am_ref[...]
        amc = jnp.where(jnp.isfinite(am), am, 0.0)
        mean = as_ref[...] / jnp.maximum(c_ref[...], 1.0)
        z = (jnp.dot(amc, wlx_ref[...], preferred_element_type=jnp.float32)
             + jnp.dot(mean, wlm_ref[...], preferred_element_type=jnp.float32)
             + jnp.dot(x_ref[...], wr_ref[...], preferred_element_type=jnp.float32)
             + b_ref[0, :][None, :])
        zm = z - jnp.max(z, axis=1, keepdims=True)
        o_ref[...] = zm - jnp.log(jnp.sum(jnp.exp(zm), axis=1, keepdims=True))

    return pl.pallas_call(
        body,
        grid=(NPAD // 256,),
        in_specs=[
            pl.BlockSpec((256, D_FEAT), lambda i: (i, 0)),
            pl.BlockSpec((256, D_FEAT), lambda i: (i, 0)),
            pl.BlockSpec((256, 1), lambda i: (i, 0)),
            pl.BlockSpec((256, D_FEAT), lambda i: (i, 0)),
            pl.BlockSpec((D_FEAT, N_CLASSES), lambda i: (0, 0)),
            pl.BlockSpec((D_FEAT, N_CLASSES), lambda i: (0, 0)),
            pl.BlockSpec((D_FEAT, N_CLASSES), lambda i: (0, 0)),
            pl.BlockSpec((8, N_CLASSES), lambda i: (0, 0)),
        ],
        out_specs=pl.BlockSpec((256, N_CLASSES), lambda i: (i, 0)),
        out_shape=jax.ShapeDtypeStruct((NPAD, N_CLASSES), jnp.float32),
    )(amax, asum, cnt, x_pad, WlmaxT, WlmeanT, WrT, bias)


def kernel(x, edge_index, Wl_max, bl_max, Wr_max, Wl_mean, bl_mean, Wr_mean):
    src = edge_index[0].astype(jnp.int32)
    dst = edge_index[1].astype(jnp.int32)
    x_pad = jnp.pad(x, ((0, NPAD - N_NODES), (0, 0)))

    amax, asum, cnt = _sc_aggregate(x_pad, src, dst)
    cnt_col = cnt.reshape(NW, CROWS * 16)[:, :RNG].reshape(NPAD, 1)

    WrT = (Wr_max + Wr_mean).T
    bias = jnp.broadcast_to((bl_max + bl_mean)[None, :], (8, N_CLASSES))
    out = _tc_head(amax, asum, cnt_col, x_pad, Wl_max.T, Wl_mean.T, WrT, bias)
    return out[:N_NODES]
